# Initial kernel scaffold; baseline (speedup 1.0000x reference)
import jax
import jax.numpy as jnp
from jax.experimental import pallas as pl


def kernel(x, edge_index, W1_l, W1_r, b1, W2_l, W2_r, b2, fc1_W, fc1_b, fc2_W, fc2_b):
    raise NotImplementedError("write your pallas kernel here")


# speedup vs baseline: 5.2346x; 5.2346x over previous
import jax
import jax.numpy as jnp
from jax import lax
from jax.experimental import pallas as pl
from jax.experimental.pallas import tpu as pltpu
from jax.experimental.pallas import tpu_sc as plsc

N = 10000
E = 320000
D_IN = 128
H = 16

NC = 2
NS = 16
NW = NC * NS

CB = 80
ROWS_TOT = E // CB
ROWS_W = ROWS_TOT // NW
EDGES_W = E // NW
N_PAD = 10240
NSTRIPE = N_PAD // NS

_MESH = plsc.VectorSubcoreMesh(
    core_axis_name="c", subcore_axis_name="s", num_cores=NC, num_subcores=NS)
_SC_PARAMS = pltpu.CompilerParams(use_tc_tiling_on_sc=False)


def _dense_in(x, wl, wr, b):
    def body(x_ref, wl_ref, wr_ref, b_ref, p_ref, r_ref):
        xb = x_ref[...]
        p_ref[...] = jnp.dot(xb, wl_ref[...], preferred_element_type=jnp.float32)
        r_ref[...] = jnp.dot(xb, wr_ref[...], preferred_element_type=jnp.float32) + b_ref[...]
    return pl.pallas_call(
        body,
        out_shape=[jax.ShapeDtypeStruct((N, H), jnp.float32)] * 2,
    )(x, wl, wr, b)


def _mid(accp, cntp, r1, wl, wr, b):
    def body(a_ref, c_ref, r1_ref, wl_ref, wr_ref, b_ref, p_ref, r_ref, inv_ref):
        acc = a_ref[0] + a_ref[1]
        cnt = c_ref[0] + c_ref[1]
        inv = 1.0 / jnp.maximum(cnt, 1.0)
        h = jnp.maximum(acc * inv + r1_ref[...], 0.0)
        p_ref[...] = jnp.dot(h, wl_ref[...], preferred_element_type=jnp.float32)
        r_ref[...] = jnp.dot(h, wr_ref[...], preferred_element_type=jnp.float32) + b_ref[...]
        inv_ref[...] = inv
    return pl.pallas_call(
        body,
        out_shape=[jax.ShapeDtypeStruct((N, H), jnp.float32)] * 3,
    )(accp, cntp, r1, wl, wr, b)


def _post(accp, inv, r2, wa, wb, b):
    def body(a_ref, inv_ref, r2_ref, wa_ref, wb_ref, b_ref, ta_ref, tb_ref):
        acc = a_ref[0] + a_ref[1]
        h = jnp.maximum(acc * inv_ref[...] + r2_ref[...], 0.0)
        ta_ref[...] = jnp.dot(h, wa_ref[...], preferred_element_type=jnp.float32) + b_ref[...]
        tb_ref[...] = jnp.dot(h, wb_ref[...], preferred_element_type=jnp.float32)
    return pl.pallas_call(
        body,
        out_shape=[jax.ShapeDtypeStruct((N, H), jnp.float32)] * 2,
    )(accp, inv, r2, wa, wb, b)


def _edge_mlp(ea, eb, w, b):
    BLK = 8000
    def body(ea_ref, eb_ref, w_ref, b_ref, o_ref):
        t = jnp.maximum(ea_ref[...] + eb_ref[...], 0.0)
        o = jnp.dot(t, w_ref[...], preferred_element_type=jnp.float32) + b_ref[...]
        m = jnp.max(o, axis=1, keepdims=True)
        s = jnp.sum(jnp.exp(o - m), axis=1, keepdims=True)
        o_ref[...] = o - m - jnp.log(s)
    return pl.pallas_call(
        body,
        grid=(E // BLK,),
        in_specs=[
            pl.BlockSpec((BLK, H), lambda i: (i, 0)),
            pl.BlockSpec((BLK, H), lambda i: (i, 0)),
            pl.BlockSpec((H, H), lambda i: (0, 0)),
            pl.BlockSpec((1, H), lambda i: (0, 0)),
        ],
        out_specs=pl.BlockSpec((BLK, H), lambda i: (i, 0)),
        out_shape=jax.ShapeDtypeStruct((E, H), jnp.float32),
    )(ea, eb, w, b)


def _agg_count(tbl, srcm, dstm, ones_rows):
    def body(tbl_hbm, srcm_hbm, dstm_hbm, ones_hbm, acc_out, cnt_out,
             idxs_v, idxd_v, rows_v, ones_v, zer_v, acc_sh, cnt_sh, sem):
        cid = lax.axis_index("c")
        sid = lax.axis_index("s")
        wid = sid * NC + cid
        pltpu.sync_copy(srcm_hbm.at[wid], idxs_v)
        pltpu.sync_copy(dstm_hbm.at[wid], idxd_v)
        pltpu.sync_copy(ones_hbm, ones_v)

        def zfill(i, carry):
            zer_v[i] = jnp.zeros((H,), jnp.float32)
            return carry
        lax.fori_loop(0, NSTRIPE, zfill, 0)
        stripe = pl.ds(sid * NSTRIPE, NSTRIPE)
        pltpu.sync_copy(zer_v, acc_sh.at[stripe])
        pltpu.sync_copy(zer_v, cnt_sh.at[stripe])
        plsc.subcore_barrier()

        def step(j, carry):
            pltpu.async_copy(tbl_hbm.at[idxs_v.at[j]], rows_v, sem).wait()
            pltpu.sync_copy(rows_v, acc_sh.at[idxd_v.at[j]], add=True)
            pltpu.sync_copy(ones_v, cnt_sh.at[idxd_v.at[j]], add=True)
            return carry
        lax.fori_loop(0, ROWS_W, step, 0)

        plsc.subcore_barrier()
        pltpu.sync_copy(acc_sh.at[stripe], acc_out.at[cid, stripe])
        pltpu.sync_copy(cnt_sh.at[stripe], cnt_out.at[cid, stripe])

    f = pl.kernel(
        body,
        out_type=[jax.ShapeDtypeStruct((NC, N_PAD, H), jnp.float32)] * 2,
        mesh=_MESH,
        compiler_params=_SC_PARAMS,
        scratch_types=[
            pltpu.VMEM((ROWS_W, CB), jnp.int32),
            pltpu.VMEM((ROWS_W, CB), jnp.int32),
            pltpu.VMEM((CB, H), jnp.float32),
            pltpu.VMEM((CB, H), jnp.float32),
            pltpu.VMEM((NSTRIPE, H), jnp.float32),
            pltpu.VMEM_SHARED((N_PAD, H), jnp.float32),
            pltpu.VMEM_SHARED((N_PAD, H), jnp.float32),
            pltpu.SemaphoreType.DMA,
        ],
    )
    return f(tbl, srcm, dstm, ones_rows)


def _agg(tbl, srcm, dstm):
    def body(tbl_hbm, srcm_hbm, dstm_hbm, acc_out,
             idxs_v, idxd_v, rows_v, zer_v, acc_sh, sem):
        cid = lax.axis_index("c")
        sid = lax.axis_index("s")
        wid = sid * NC + cid
        pltpu.sync_copy(srcm_hbm.at[wid], idxs_v)
        pltpu.sync_copy(dstm_hbm.at[wid], idxd_v)

        def zfill(i, carry):
            zer_v[i] = jnp.zeros((H,), jnp.float32)
            return carry
        lax.fori_loop(0, NSTRIPE, zfill, 0)
        stripe = pl.ds(sid * NSTRIPE, NSTRIPE)
        pltpu.sync_copy(zer_v, acc_sh.at[stripe])
        plsc.subcore_barrier()

        def step(j, carry):
            pltpu.async_copy(tbl_hbm.at[idxs_v.at[j]], rows_v, sem).wait()
            pltpu.sync_copy(rows_v, acc_sh.at[idxd_v.at[j]], add=True)
            return carry
        lax.fori_loop(0, ROWS_W, step, 0)

        plsc.subcore_barrier()
        pltpu.sync_copy(acc_sh.at[stripe], acc_out.at[cid, stripe])

    f = pl.kernel(
        body,
        out_type=jax.ShapeDtypeStruct((NC, N_PAD, H), jnp.float32),
        mesh=_MESH,
        compiler_params=_SC_PARAMS,
        scratch_types=[
            pltpu.VMEM((ROWS_W, CB), jnp.int32),
            pltpu.VMEM((ROWS_W, CB), jnp.int32),
            pltpu.VMEM((CB, H), jnp.float32),
            pltpu.VMEM((NSTRIPE, H), jnp.float32),
            pltpu.VMEM_SHARED((N_PAD, H), jnp.float32),
            pltpu.SemaphoreType.DMA,
        ],
    )
    return f(tbl, srcm, dstm)


def _edge_gather(ta, tb, srcm, dstm):
    def body(ta_hbm, tb_hbm, srcm_hbm, dstm_hbm, ea_out, eb_out,
             idxs_v, idxd_v, rowsa_v, rowsb_v, sema, semb):
        cid = lax.axis_index("c")
        sid = lax.axis_index("s")
        wid = sid * NC + cid
        pltpu.sync_copy(srcm_hbm.at[wid], idxs_v)
        pltpu.sync_copy(dstm_hbm.at[wid], idxd_v)
        base_e = wid * EDGES_W

        def step(j, carry):
            ca = pltpu.async_copy(ta_hbm.at[idxs_v.at[j]], rowsa_v, sema)
            cb = pltpu.async_copy(tb_hbm.at[idxd_v.at[j]], rowsb_v, semb)
            ca.wait()
            cb.wait()
            off = base_e + j * CB
            pltpu.sync_copy(rowsa_v, ea_out.at[pl.ds(off, CB)])
            pltpu.sync_copy(rowsb_v, eb_out.at[pl.ds(off, CB)])
            return carry
        lax.fori_loop(0, ROWS_W, step, 0)

    f = pl.kernel(
        body,
        out_type=[jax.ShapeDtypeStruct((E, H), jnp.float32)] * 2,
        mesh=_MESH,
        compiler_params=_SC_PARAMS,
        scratch_types=[
            pltpu.VMEM((ROWS_W, CB), jnp.int32),
            pltpu.VMEM((ROWS_W, CB), jnp.int32),
            pltpu.VMEM((CB, H), jnp.float32),
            pltpu.VMEM((CB, H), jnp.float32),
            pltpu.SemaphoreType.DMA,
            pltpu.SemaphoreType.DMA,
        ],
    )
    return f(ta, tb, srcm, dstm)


def kernel(x, edge_index, W1_l, W1_r, b1, W2_l, W2_r, b2, fc1_W, fc1_b, fc2_W, fc2_b):
    src = edge_index[0].astype(jnp.int32)
    dst = edge_index[1].astype(jnp.int32)
    srcm = src.reshape(NW, ROWS_W, CB)
    dstm = dst.reshape(NW, ROWS_W, CB)
    ones_rows = jnp.ones((CB, H), jnp.float32)

    p1, r1 = _dense_in(x, W1_l, W1_r, b1.reshape(1, H))
    accp1, cntp = _agg_count(p1, srcm, dstm, ones_rows)
    p2, r2, inv = _mid(accp1[:, :N], cntp[:, :N], r1, W2_l, W2_r, b2.reshape(1, H))
    accp2 = _agg(p2, srcm, dstm)
    ta, tb = _post(accp2[:, :N], inv, r2, fc1_W[:H], fc1_W[H:], fc1_b.reshape(1, H))
    ea, eb = _edge_gather(ta, tb, srcm, dstm)
    return _edge_mlp(ea, eb, fc2_W, fc2_b.reshape(1, H))


# speedup vs baseline: 5.6806x; 1.0852x over previous
import jax
import jax.numpy as jnp
from jax import lax
from jax.experimental import pallas as pl
from jax.experimental.pallas import tpu as pltpu
from jax.experimental.pallas import tpu_sc as plsc

N = 10000
E = 320000
D_IN = 128
H = 16

NC = 2
NS = 16
NW = NC * NS

CB = 80
ROWS_TOT = E // CB
ROWS_W = ROWS_TOT // NW
EDGES_W = E // NW
NBUF = 5
NOUTER = ROWS_W // NBUF
N_PAD = 10240
NSTRIPE = N_PAD // NS

_MESH = plsc.VectorSubcoreMesh(
    core_axis_name="c", subcore_axis_name="s", num_cores=NC, num_subcores=NS)
_SC_PARAMS = pltpu.CompilerParams(use_tc_tiling_on_sc=False)


def _dense_in(x, wl, wr, b):
    def body(x_ref, wl_ref, wr_ref, b_ref, p_ref, r_ref):
        xb = x_ref[...]
        p_ref[...] = jnp.dot(xb, wl_ref[...], preferred_element_type=jnp.float32)
        r_ref[...] = jnp.dot(xb, wr_ref[...], preferred_element_type=jnp.float32) + b_ref[...]
    return pl.pallas_call(
        body,
        out_shape=[jax.ShapeDtypeStruct((N, H), jnp.float32)] * 2,
    )(x, wl, wr, b)


def _mid(accp, cntp, r1, wl, wr, b):
    def body(a_ref, c_ref, r1_ref, wl_ref, wr_ref, b_ref, p_ref, r_ref, inv_ref):
        acc = a_ref[0] + a_ref[1]
        cnt = c_ref[0] + c_ref[1]
        inv = 1.0 / jnp.maximum(cnt, 1.0)
        h = jnp.maximum(acc * inv + r1_ref[...], 0.0)
        p_ref[...] = jnp.dot(h, wl_ref[...], preferred_element_type=jnp.float32)
        r_ref[...] = jnp.dot(h, wr_ref[...], preferred_element_type=jnp.float32) + b_ref[...]
        inv_ref[...] = inv
    return pl.pallas_call(
        body,
        out_shape=[jax.ShapeDtypeStruct((N, H), jnp.float32)] * 3,
    )(accp, cntp, r1, wl, wr, b)


def _post(accp, inv, r2, wa, wb, b):
    def body(a_ref, inv_ref, r2_ref, wa_ref, wb_ref, b_ref, ta_ref, tb_ref):
        acc = a_ref[0] + a_ref[1]
        h = jnp.maximum(acc * inv_ref[...] + r2_ref[...], 0.0)
        ta_ref[...] = jnp.dot(h, wa_ref[...], preferred_element_type=jnp.float32) + b_ref[...]
        tb_ref[...] = jnp.dot(h, wb_ref[...], preferred_element_type=jnp.float32)
    return pl.pallas_call(
        body,
        out_shape=[jax.ShapeDtypeStruct((N, H), jnp.float32)] * 2,
    )(accp, inv, r2, wa, wb, b)


def _edge_mlp(ea, eb, w, b):
    BLK = 8000
    def body(ea_ref, eb_ref, w_ref, b_ref, o_ref):
        t = jnp.maximum(ea_ref[...] + eb_ref[...], 0.0)
        o = jnp.dot(t, w_ref[...], preferred_element_type=jnp.float32) + b_ref[...]
        m = jnp.max(o, axis=1, keepdims=True)
        s = jnp.sum(jnp.exp(o - m), axis=1, keepdims=True)
        o_ref[...] = o - m - jnp.log(s)
    return pl.pallas_call(
        body,
        grid=(E // BLK,),
        in_specs=[
            pl.BlockSpec((BLK, H), lambda i: (i, 0)),
            pl.BlockSpec((BLK, H), lambda i: (i, 0)),
            pl.BlockSpec((H, H), lambda i: (0, 0)),
            pl.BlockSpec((1, H), lambda i: (0, 0)),
        ],
        out_specs=pl.BlockSpec((BLK, H), lambda i: (i, 0)),
        out_shape=jax.ShapeDtypeStruct((E, H), jnp.float32),
    )(ea, eb, w, b)


def _agg_impl(with_count, tbl, srcm, dstm, ones_rows):
    def body(*refs):
        if with_count:
            (tbl_hbm, srcm_hbm, dstm_hbm, ones_hbm, acc_out, cnt_out,
             idxs_v, idxd_v, rows_v, ones_v, zer_v, acc_sh, cnt_sh,
             sem_g, sem_a, sem_c) = refs
        else:
            (tbl_hbm, srcm_hbm, dstm_hbm, ones_hbm, acc_out,
             idxs_v, idxd_v, rows_v, ones_v, zer_v, acc_sh,
             sem_g, sem_a, sem_c) = refs
        cid = lax.axis_index("c")
        sid = lax.axis_index("s")
        wid = sid * NC + cid
        pltpu.sync_copy(srcm_hbm.at[wid], idxs_v.at[pl.ds(0, ROWS_W)])
        pltpu.sync_copy(dstm_hbm.at[wid], idxd_v)
        pltpu.sync_copy(ones_hbm, ones_v)
        for k in range(NBUF):
            for i in range(CB // 16):
                idxs_v[ROWS_W + k, pl.ds(i * 16, 16)] = jnp.zeros((16,), jnp.int32)

        def zfill(i, carry):
            zer_v[i] = jnp.zeros((H,), jnp.float32)
            return carry
        lax.fori_loop(0, NSTRIPE, zfill, 0)
        stripe = pl.ds(sid * NSTRIPE, NSTRIPE)
        pltpu.sync_copy(zer_v, acc_sh.at[stripe])
        if with_count:
            pltpu.sync_copy(zer_v, cnt_sh.at[stripe])
        plsc.subcore_barrier()

        for b in range(NBUF - 1):
            pltpu.async_copy(tbl_hbm.at[idxs_v.at[b]], rows_v.at[b], sem_g)

        def outer(jo, carry):
            j0 = jo * NBUF
            for b in range(NBUF):
                j = j0 + b
                if b == 0:
                    @pl.when(jo > 0)
                    def _():
                        pltpu.make_async_copy(ones_hbm, rows_v.at[NBUF - 1], sem_a).wait()
                        if with_count:
                            pltpu.make_async_copy(ones_hbm, ones_v, sem_c).wait()
                else:
                    pltpu.make_async_copy(ones_hbm, rows_v.at[b - 1], sem_a).wait()
                    if with_count:
                        pltpu.make_async_copy(ones_hbm, ones_v, sem_c).wait()
                pltpu.make_async_copy(ones_hbm, rows_v.at[b], sem_g).wait()
                pltpu.async_copy(rows_v.at[b], acc_sh.at[idxd_v.at[j]], sem_a, add=True)
                if with_count:
                    pltpu.async_copy(ones_v, cnt_sh.at[idxd_v.at[j]], sem_c, add=True)
                pltpu.async_copy(tbl_hbm.at[idxs_v.at[j + NBUF - 1]],
                                 rows_v.at[(b - 1) % NBUF], sem_g)
            return carry
        lax.fori_loop(0, NOUTER, outer, 0)

        pltpu.make_async_copy(ones_hbm, rows_v.at[0], sem_a).wait()
        if with_count:
            pltpu.make_async_copy(ones_hbm, ones_v, sem_c).wait()
        for b in range(NBUF - 1):
            pltpu.make_async_copy(ones_hbm, rows_v.at[b], sem_g).wait()

        plsc.subcore_barrier()
        pltpu.sync_copy(acc_sh.at[stripe], acc_out.at[cid, stripe])
        if with_count:
            pltpu.sync_copy(cnt_sh.at[stripe], cnt_out.at[cid, stripe])

    ot = jax.ShapeDtypeStruct((NC, N_PAD, H), jnp.float32)
    shared = [pltpu.VMEM_SHARED((N_PAD, H), jnp.float32)] * (2 if with_count else 1)
    f = pl.kernel(
        body,
        out_type=[ot, ot] if with_count else ot,
        mesh=_MESH,
        compiler_params=_SC_PARAMS,
        scratch_types=[
            pltpu.VMEM((ROWS_W + NBUF, CB), jnp.int32),
            pltpu.VMEM((ROWS_W, CB), jnp.int32),
            pltpu.VMEM((NBUF, CB, H), jnp.float32),
            pltpu.VMEM((CB, H), jnp.float32),
            pltpu.VMEM((NSTRIPE, H), jnp.float32),
        ] + shared + [
            pltpu.SemaphoreType.DMA,
            pltpu.SemaphoreType.DMA,
            pltpu.SemaphoreType.DMA,
        ],
    )
    out = f(tbl, srcm, dstm, ones_rows)
    return out if with_count else (out,)


def _edge_gather(ta, tb, srcm, dstm):
    def body(ta_hbm, tb_hbm, srcm_hbm, dstm_hbm, ea_out, eb_out,
             idxs_v, idxd_v, rowsa_v, rowsb_v, sem_ga, sem_gb, sem_w):
        cid = lax.axis_index("c")
        sid = lax.axis_index("s")
        wid = sid * NC + cid
        pltpu.sync_copy(srcm_hbm.at[wid], idxs_v.at[pl.ds(0, ROWS_W)])
        pltpu.sync_copy(dstm_hbm.at[wid], idxd_v.at[pl.ds(0, ROWS_W)])
        for k in range(NBUF):
            for i in range(CB // 16):
                idxs_v[ROWS_W + k, pl.ds(i * 16, 16)] = jnp.zeros((16,), jnp.int32)
                idxd_v[ROWS_W + k, pl.ds(i * 16, 16)] = jnp.zeros((16,), jnp.int32)
        base_e = wid * EDGES_W

        for b in range(NBUF - 1):
            pltpu.async_copy(ta_hbm.at[idxs_v.at[b]], rowsa_v.at[b], sem_ga)
            pltpu.async_copy(tb_hbm.at[idxd_v.at[b]], rowsb_v.at[b], sem_gb)

        def outer(jo, carry):
            j0 = jo * NBUF
            for b in range(NBUF):
                j = j0 + b
                if b == 0:
                    @pl.when(jo > 0)
                    def _():
                        pltpu.make_async_copy(ta_hbm.at[pl.ds(0, CB)], rowsa_v.at[NBUF - 1], sem_w).wait()
                        pltpu.make_async_copy(ta_hbm.at[pl.ds(0, CB)], rowsb_v.at[NBUF - 1], sem_w).wait()
                else:
                    pltpu.make_async_copy(ta_hbm.at[pl.ds(0, CB)], rowsa_v.at[b - 1], sem_w).wait()
                    pltpu.make_async_copy(ta_hbm.at[pl.ds(0, CB)], rowsb_v.at[b - 1], sem_w).wait()
                pltpu.make_async_copy(ta_hbm.at[pl.ds(0, CB)], rowsa_v.at[b], sem_ga).wait()
                pltpu.make_async_copy(ta_hbm.at[pl.ds(0, CB)], rowsb_v.at[b], sem_gb).wait()
                off = base_e + j * CB
                pltpu.async_copy(rowsa_v.at[b], ea_out.at[pl.ds(off, CB)], sem_w)
                pltpu.async_copy(rowsb_v.at[b], eb_out.at[pl.ds(off, CB)], sem_w)
                pltpu.async_copy(ta_hbm.at[idxs_v.at[j + NBUF - 1]],
                                 rowsa_v.at[(b - 1) % NBUF], sem_ga)
                pltpu.async_copy(tb_hbm.at[idxd_v.at[j + NBUF - 1]],
                                 rowsb_v.at[(b - 1) % NBUF], sem_gb)
            return carry
        lax.fori_loop(0, NOUTER, outer, 0)

        pltpu.make_async_copy(ta_hbm.at[pl.ds(0, CB)], rowsa_v.at[0], sem_w).wait()
        pltpu.make_async_copy(ta_hbm.at[pl.ds(0, CB)], rowsb_v.at[0], sem_w).wait()
        for b in range(NBUF - 1):
            pltpu.make_async_copy(ta_hbm.at[pl.ds(0, CB)], rowsa_v.at[b], sem_ga).wait()
            pltpu.make_async_copy(ta_hbm.at[pl.ds(0, CB)], rowsb_v.at[b], sem_gb).wait()

    f = pl.kernel(
        body,
        out_type=[jax.ShapeDtypeStruct((E, H), jnp.float32)] * 2,
        mesh=_MESH,
        compiler_params=_SC_PARAMS,
        scratch_types=[
            pltpu.VMEM((ROWS_W + NBUF, CB), jnp.int32),
            pltpu.VMEM((ROWS_W + NBUF, CB), jnp.int32),
            pltpu.VMEM((NBUF, CB, H), jnp.float32),
            pltpu.VMEM((NBUF, CB, H), jnp.float32),
            pltpu.SemaphoreType.DMA,
            pltpu.SemaphoreType.DMA,
            pltpu.SemaphoreType.DMA,
        ],
    )
    return f(ta, tb, srcm, dstm)


def kernel(x, edge_index, W1_l, W1_r, b1, W2_l, W2_r, b2, fc1_W, fc1_b, fc2_W, fc2_b):
    src = edge_index[0].astype(jnp.int32)
    dst = edge_index[1].astype(jnp.int32)
    srcm = src.reshape(NW, ROWS_W, CB)
    dstm = dst.reshape(NW, ROWS_W, CB)
    ones_rows = jnp.ones((CB, H), jnp.float32)

    p1, r1 = _dense_in(x, W1_l, W1_r, b1.reshape(1, H))
    accp1, cntp = _agg_impl(True, p1, srcm, dstm, ones_rows)
    p2, r2, inv = _mid(accp1[:, :N], cntp[:, :N], r1, W2_l, W2_r, b2.reshape(1, H))
    accp2, = _agg_impl(False, p2, srcm, dstm, ones_rows)
    ta, tb = _post(accp2[:, :N], inv, r2, fc1_W[:H], fc1_W[H:], fc1_b.reshape(1, H))
    ea, eb = _edge_gather(ta, tb, srcm, dstm)
    return _edge_mlp(ea, eb, fc2_W, fc2_b.reshape(1, H))
